# Initial kernel scaffold; baseline (speedup 1.0000x reference)
import functools

import jax
import jax.numpy as jnp
from jax import lax
from jax.experimental import pallas as pl
from jax.experimental.pallas import tpu as pltpu
from jax.experimental.pallas import tpu_sc as plsc

N = 10000
E = 320000
F = 128
C = 40

N_PAD = 10240
NW = 32
CHUNK = 128
CH = 80
EPW = CH * CHUNK
E_PAD = NW * EPW
ROWS_PT = N_PAD // 16

_BLK = 512
_GRID = N_PAD // _BLK


def _sc_mesh():
    return plsc.VectorSubcoreMesh(core_axis_name="c", subcore_axis_name="s")


def _deg_body(dst_hbm, ones_hbm, zeros_hbm, out_hbm, shared, idx_v, ones_v):
    cid = lax.axis_index("c")
    sid = lax.axis_index("s")
    wid = cid * 16 + sid
    r0 = sid * ROWS_PT
    pltpu.sync_copy(zeros_hbm.at[pl.ds(r0, ROWS_PT)], shared.at[pl.ds(r0, ROWS_PT)])
    pltpu.sync_copy(dst_hbm.at[pl.ds(wid * CH, CH)], idx_v)
    pltpu.sync_copy(ones_hbm, ones_v)
    plsc.subcore_barrier()

    @pl.loop(0, CH)
    def _(j):
        pltpu.sync_copy(ones_v, shared.at[idx_v.at[j]], add=True)

    plsc.subcore_barrier()
    pltpu.sync_copy(shared.at[pl.ds(r0, ROWS_PT)],
                    out_hbm.at[cid, pl.ds(r0, ROWS_PT)])


def _sc_degree(dst2d, ones16, zeros16):
    return pl.kernel(
        _deg_body,
        out_type=jax.ShapeDtypeStruct((2, N_PAD, 16), jnp.float32),
        mesh=_sc_mesh(),
        scratch_types=[
            pltpu.VMEM_SHARED((N_PAD, 16), jnp.float32),
            pltpu.VMEM((CH, CHUNK), jnp.int32),
            pltpu.VMEM((CHUNK, 16), jnp.float32),
        ],
    )(dst2d, ones16, zeros16)


def _agg_body(y_hbm, src_hbm, dst_hbm, zeros_hbm, out_hbm,
              shared, sidx, didx, rows, sem):
    cid = lax.axis_index("c")
    sid = lax.axis_index("s")
    wid = cid * 16 + sid
    r0 = sid * ROWS_PT
    pltpu.sync_copy(zeros_hbm.at[pl.ds(r0, ROWS_PT)], shared.at[pl.ds(r0, ROWS_PT)])
    pltpu.sync_copy(src_hbm.at[pl.ds(wid * CH, CH)], sidx)
    pltpu.sync_copy(dst_hbm.at[pl.ds(wid * CH, CH)], didx)
    plsc.subcore_barrier()

    @pl.loop(0, CH)
    def _(j):
        pltpu.async_copy(y_hbm.at[sidx.at[j]], rows, sem).wait()
        pltpu.sync_copy(rows, shared.at[didx.at[j]], add=True)

    plsc.subcore_barrier()
    pltpu.sync_copy(shared.at[pl.ds(r0, ROWS_PT)],
                    out_hbm.at[cid, pl.ds(r0, ROWS_PT)])


def _sc_agg(y, src2d, dst2d, zeros128):
    return pl.kernel(
        _agg_body,
        out_type=jax.ShapeDtypeStruct((2, N_PAD, F), jnp.float32),
        mesh=_sc_mesh(),
        scratch_types=[
            pltpu.VMEM_SHARED((N_PAD, F), jnp.float32),
            pltpu.VMEM((CH, CHUNK), jnp.int32),
            pltpu.VMEM((CH, CHUNK), jnp.int32),
            pltpu.VMEM((CHUNK, F), jnp.float32),
            pltpu.SemaphoreType.DMA,
        ],
    )(y, src2d, dst2d, zeros128)


def _prep_body(deg_ref, x_ref, dis_ref, y_ref):
    deg = deg_ref[0, :, 0:1] + deg_ref[1, :, 0:1] + 1.0
    dis = lax.rsqrt(deg)
    dis_ref[...] = jnp.broadcast_to(dis, (_BLK, 8))
    y_ref[...] = x_ref[...] * dis


def _tc_prep(deg_tbl, x_pad):
    return pl.pallas_call(
        _prep_body,
        grid=(_GRID,),
        in_specs=[
            pl.BlockSpec((2, _BLK, 16), lambda i: (0, i, 0)),
            pl.BlockSpec((_BLK, F), lambda i: (i, 0)),
        ],
        out_specs=[
            pl.BlockSpec((_BLK, 8), lambda i: (i, 0)),
            pl.BlockSpec((_BLK, F), lambda i: (i, 0)),
        ],
        out_shape=[
            jax.ShapeDtypeStruct((N_PAD, 8), jnp.float32),
            jax.ShapeDtypeStruct((N_PAD, F), jnp.float32),
        ],
    )(deg_tbl, x_pad)


def _layer_body(agg_ref, y_ref, dis_ref, w_ref, sc_ref, sh_ref, out_ref):
    dis = dis_ref[:, 0:1]
    z = (agg_ref[0] + agg_ref[1] + y_ref[...]) * dis
    h = jnp.dot(z, w_ref[...], preferred_element_type=jnp.float32)
    h = h * sc_ref[...] + sh_ref[...]
    out_ref[...] = jnp.maximum(h, 0.0) * dis


def _tc_layer(agg, y, dis8, w, scale, shift):
    return pl.pallas_call(
        _layer_body,
        grid=(_GRID,),
        in_specs=[
            pl.BlockSpec((2, _BLK, F), lambda i: (0, i, 0)),
            pl.BlockSpec((_BLK, F), lambda i: (i, 0)),
            pl.BlockSpec((_BLK, 8), lambda i: (i, 0)),
            pl.BlockSpec((F, F), lambda i: (0, 0)),
            pl.BlockSpec((1, F), lambda i: (0, 0)),
            pl.BlockSpec((1, F), lambda i: (0, 0)),
        ],
        out_specs=pl.BlockSpec((_BLK, F), lambda i: (i, 0)),
        out_shape=jax.ShapeDtypeStruct((N_PAD, F), jnp.float32),
    )(agg, y, dis8, w, scale, shift)


def _final_body(agg_ref, y_ref, dis_ref, w_ref, b_ref, out_ref):
    dis = dis_ref[:, 0:1]
    z = (agg_ref[0] + agg_ref[1] + y_ref[...]) * dis
    o = jnp.dot(z, w_ref[...], preferred_element_type=jnp.float32) + b_ref[...]
    m = jnp.max(o, axis=-1, keepdims=True)
    e = o - m
    lse = jnp.log(jnp.sum(jnp.exp(e), axis=-1, keepdims=True))
    out_ref[...] = e - lse


def _tc_final(agg, y, dis8, w, b):
    return pl.pallas_call(
        _final_body,
        grid=(_GRID,),
        in_specs=[
            pl.BlockSpec((2, _BLK, F), lambda i: (0, i, 0)),
            pl.BlockSpec((_BLK, F), lambda i: (i, 0)),
            pl.BlockSpec((_BLK, 8), lambda i: (i, 0)),
            pl.BlockSpec((F, C), lambda i: (0, 0)),
            pl.BlockSpec((1, C), lambda i: (0, 0)),
        ],
        out_specs=pl.BlockSpec((_BLK, C), lambda i: (i, 0)),
        out_shape=jax.ShapeDtypeStruct((N_PAD, C), jnp.float32),
    )(agg, y, dis8, w, b)


def kernel(x, adj_t, W1, b1, g1, be1, W2, b2, g2, be2, W3, b3):
    x_pad = jnp.zeros((N_PAD, F), jnp.float32).at[:N].set(x)
    pad = jnp.full((E_PAD - E,), N, dtype=jnp.int32)
    src2d = jnp.concatenate([adj_t[0], pad]).reshape(E_PAD // CHUNK, CHUNK)
    dst2d = jnp.concatenate([adj_t[1], pad]).reshape(E_PAD // CHUNK, CHUNK)
    zeros16 = jnp.zeros((N_PAD, 16), jnp.float32)
    ones16 = jnp.ones((CHUNK, 16), jnp.float32)
    zeros128 = jnp.zeros((N_PAD, F), jnp.float32)

    k = 1.0 / jnp.sqrt(jnp.float32(1.0 + 1e-5))
    sc1 = (g1 * k).reshape(1, F)
    sh1 = (b1 * g1 * k + be1).reshape(1, F)
    sc2 = (g2 * k).reshape(1, F)
    sh2 = (b2 * g2 * k + be2).reshape(1, F)

    deg_tbl = _sc_degree(dst2d, ones16, zeros16)
    dis8, y0 = _tc_prep(deg_tbl, x_pad)
    a1 = _sc_agg(y0, src2d, dst2d, zeros128)
    y1 = _tc_layer(a1, y0, dis8, W1, sc1, sh1)
    a2 = _sc_agg(y1, src2d, dst2d, zeros128)
    y2 = _tc_layer(a2, y1, dis8, W2, sc2, sh2)
    a3 = _sc_agg(y2, src2d, dst2d, zeros128)
    out = _tc_final(a3, y2, dis8, W3, b3.reshape(1, C))
    return out[:N]


# speedup vs baseline: 6.2032x; 6.2032x over previous
import functools

import jax
import jax.numpy as jnp
from jax import lax
from jax.experimental import pallas as pl
from jax.experimental.pallas import tpu as pltpu
from jax.experimental.pallas import tpu_sc as plsc

N = 10000
E = 320000
F = 128
C = 40

N_PAD = 10240
NW = 32
CHUNK = 128
CH = 80
EPW = CH * CHUNK
E_PAD = NW * EPW
ROWS_PT = N_PAD // 16

_BLK = 512
_GRID = N_PAD // _BLK


def _sc_mesh():
    return plsc.VectorSubcoreMesh(core_axis_name="c", subcore_axis_name="s")


def _deg_body(dst_hbm, ones_hbm, zeros_hbm, out_hbm, shared, idx_v, ones_v):
    cid = lax.axis_index("c")
    sid = lax.axis_index("s")
    wid = cid * 16 + sid
    r0 = sid * ROWS_PT
    pltpu.sync_copy(zeros_hbm.at[pl.ds(r0, ROWS_PT)], shared.at[pl.ds(r0, ROWS_PT)])
    pltpu.sync_copy(dst_hbm.at[pl.ds(wid * CH, CH)], idx_v)
    pltpu.sync_copy(ones_hbm, ones_v)
    plsc.subcore_barrier()

    @pl.loop(0, CH)
    def _(j):
        pltpu.sync_copy(ones_v, shared.at[idx_v.at[j]], add=True)

    plsc.subcore_barrier()
    pltpu.sync_copy(shared.at[pl.ds(r0, ROWS_PT)],
                    out_hbm.at[cid, pl.ds(r0, ROWS_PT)])


def _sc_degree(dst2d, ones128, zeros128):
    return pl.kernel(
        _deg_body,
        out_type=jax.ShapeDtypeStruct((2, N_PAD, F), jnp.float32),
        mesh=_sc_mesh(),
        scratch_types=[
            pltpu.VMEM_SHARED((N_PAD, F), jnp.float32),
            pltpu.VMEM((CH, CHUNK), jnp.int32),
            pltpu.VMEM((CHUNK, F), jnp.float32),
        ],
    )(dst2d, ones128, zeros128)


def _agg_body(y_hbm, src_hbm, dst_hbm, zeros_hbm, out_hbm,
              shared, sidx, didx, rows, sem):
    cid = lax.axis_index("c")
    sid = lax.axis_index("s")
    wid = cid * 16 + sid
    r0 = sid * ROWS_PT
    pltpu.sync_copy(zeros_hbm.at[pl.ds(r0, ROWS_PT)], shared.at[pl.ds(r0, ROWS_PT)])
    pltpu.sync_copy(src_hbm.at[pl.ds(wid * CH, CH)], sidx)
    pltpu.sync_copy(dst_hbm.at[pl.ds(wid * CH, CH)], didx)
    plsc.subcore_barrier()

    @pl.loop(0, CH)
    def _(j):
        pltpu.async_copy(y_hbm.at[sidx.at[j]], rows, sem).wait()
        pltpu.sync_copy(rows, shared.at[didx.at[j]], add=True)

    plsc.subcore_barrier()
    pltpu.sync_copy(shared.at[pl.ds(r0, ROWS_PT)],
                    out_hbm.at[cid, pl.ds(r0, ROWS_PT)])


def _sc_agg(y, src2d, dst2d, zeros128):
    return pl.kernel(
        _agg_body,
        out_type=jax.ShapeDtypeStruct((2, N_PAD, F), jnp.float32),
        mesh=_sc_mesh(),
        scratch_types=[
            pltpu.VMEM_SHARED((N_PAD, F), jnp.float32),
            pltpu.VMEM((CH, CHUNK), jnp.int32),
            pltpu.VMEM((CH, CHUNK), jnp.int32),
            pltpu.VMEM((CHUNK, F), jnp.float32),
            pltpu.SemaphoreType.DMA,
        ],
    )(y, src2d, dst2d, zeros128)


def _prep_body(deg_ref, x_ref, dis_ref, y_ref):
    deg = deg_ref[0, :, 0:1] + deg_ref[1, :, 0:1] + 1.0
    dis = lax.rsqrt(deg)
    dis_ref[...] = jnp.broadcast_to(dis, (_BLK, 8))
    y_ref[...] = x_ref[...] * dis


def _tc_prep(deg_tbl, x_pad):
    return pl.pallas_call(
        _prep_body,
        grid=(_GRID,),
        in_specs=[
            pl.BlockSpec((2, _BLK, F), lambda i: (0, i, 0)),
            pl.BlockSpec((_BLK, F), lambda i: (i, 0)),
        ],
        out_specs=[
            pl.BlockSpec((_BLK, 8), lambda i: (i, 0)),
            pl.BlockSpec((_BLK, F), lambda i: (i, 0)),
        ],
        out_shape=[
            jax.ShapeDtypeStruct((N_PAD, 8), jnp.float32),
            jax.ShapeDtypeStruct((N_PAD, F), jnp.float32),
        ],
    )(deg_tbl, x_pad)


def _layer_body(agg_ref, y_ref, dis_ref, w_ref, sc_ref, sh_ref, out_ref):
    dis = dis_ref[:, 0:1]
    z = (agg_ref[0] + agg_ref[1] + y_ref[...]) * dis
    h = jnp.dot(z, w_ref[...], preferred_element_type=jnp.float32)
    h = h * sc_ref[...] + sh_ref[...]
    out_ref[...] = jnp.maximum(h, 0.0) * dis


def _tc_layer(agg, y, dis8, w, scale, shift):
    return pl.pallas_call(
        _layer_body,
        grid=(_GRID,),
        in_specs=[
            pl.BlockSpec((2, _BLK, F), lambda i: (0, i, 0)),
            pl.BlockSpec((_BLK, F), lambda i: (i, 0)),
            pl.BlockSpec((_BLK, 8), lambda i: (i, 0)),
            pl.BlockSpec((F, F), lambda i: (0, 0)),
            pl.BlockSpec((1, F), lambda i: (0, 0)),
            pl.BlockSpec((1, F), lambda i: (0, 0)),
        ],
        out_specs=pl.BlockSpec((_BLK, F), lambda i: (i, 0)),
        out_shape=jax.ShapeDtypeStruct((N_PAD, F), jnp.float32),
    )(agg, y, dis8, w, scale, shift)


def _final_body(agg_ref, y_ref, dis_ref, w_ref, b_ref, out_ref):
    dis = dis_ref[:, 0:1]
    z = (agg_ref[0] + agg_ref[1] + y_ref[...]) * dis
    o = jnp.dot(z, w_ref[...], preferred_element_type=jnp.float32) + b_ref[...]
    m = jnp.max(o, axis=-1, keepdims=True)
    e = o - m
    lse = jnp.log(jnp.sum(jnp.exp(e), axis=-1, keepdims=True))
    out_ref[...] = e - lse


def _tc_final(agg, y, dis8, w, b):
    return pl.pallas_call(
        _final_body,
        grid=(_GRID,),
        in_specs=[
            pl.BlockSpec((2, _BLK, F), lambda i: (0, i, 0)),
            pl.BlockSpec((_BLK, F), lambda i: (i, 0)),
            pl.BlockSpec((_BLK, 8), lambda i: (i, 0)),
            pl.BlockSpec((F, C), lambda i: (0, 0)),
            pl.BlockSpec((1, C), lambda i: (0, 0)),
        ],
        out_specs=pl.BlockSpec((_BLK, C), lambda i: (i, 0)),
        out_shape=jax.ShapeDtypeStruct((N_PAD, C), jnp.float32),
    )(agg, y, dis8, w, b)


def kernel(x, adj_t, W1, b1, g1, be1, W2, b2, g2, be2, W3, b3):
    x_pad = jnp.zeros((N_PAD, F), jnp.float32).at[:N].set(x)
    pad = jnp.full((E_PAD - E,), N, dtype=jnp.int32)
    src2d = jnp.concatenate([adj_t[0], pad]).reshape(E_PAD // CHUNK, CHUNK)
    dst2d = jnp.concatenate([adj_t[1], pad]).reshape(E_PAD // CHUNK, CHUNK)
    ones128 = jnp.ones((CHUNK, F), jnp.float32)
    zeros128 = jnp.zeros((N_PAD, F), jnp.float32)

    k = 1.0 / jnp.sqrt(jnp.float32(1.0 + 1e-5))
    sc1 = (g1 * k).reshape(1, F)
    sh1 = (b1 * g1 * k + be1).reshape(1, F)
    sc2 = (g2 * k).reshape(1, F)
    sh2 = (b2 * g2 * k + be2).reshape(1, F)

    deg_tbl = _sc_degree(dst2d, ones128, zeros128)
    dis8, y0 = _tc_prep(deg_tbl, x_pad)
    a1 = _sc_agg(y0, src2d, dst2d, zeros128)
    y1 = _tc_layer(a1, y0, dis8, W1, sc1, sh1)
    a2 = _sc_agg(y1, src2d, dst2d, zeros128)
    y2 = _tc_layer(a2, y1, dis8, W2, sc2, sh2)
    a3 = _sc_agg(y2, src2d, dst2d, zeros128)
    out = _tc_final(a3, y2, dis8, W3, b3.reshape(1, C))
    return out[:N]
